# baseline (speedup 1.0000x reference)
import jax
import jax.numpy as jnp
from jax import lax
from jax.experimental import pallas as pl
from jax.experimental.pallas import tpu as pltpu
from jax.experimental.pallas import tpu_sc as plsc

NUM_TILES = 1000000
D_MODEL = 64
BATCH = 16384

_NW = 32
_W = 128
_FULL_CHUNKS = 7812
_TAIL_LO = _FULL_CHUNKS * _W
_NTAIL = NUM_TILES - _TAIL_LO
_WCAP = 960
_SN = 1040
_MCAP = 32
_PIECE = 4096
_NPIECE = BATCH // _PIECE


def _body(x_hbm, idx_hbm, ts_hbm, th_hbm, tails_hbm, tailh_hbm, out_hbm,
          idx_p, wb, wi, skey, tbs, tbh, tails_v, tailh_v, xbank, obuf,
          xrow_fb, semt0, semt1, semx0, semx1, semo0, semo1):
    wid = lax.axis_index("s") * 2 + lax.axis_index("c")
    iota = lax.iota(jnp.int32, 16)

    def splat(v):
        return jnp.full((16,), v, jnp.int32)

    def fire_tables(k, s, semt):
        lo = (wid + _NW * k) * _W
        pltpu.async_copy(ts_hbm.at[:, pl.ds(lo, _W)], tbs.at[s], semt)
        pltpu.async_copy(th_hbm.at[:, pl.ds(lo, _W)], tbh.at[s], semt)

    def wait_tables(s, semt):
        pltpu.make_async_copy(ts_hbm.at[:, pl.ds(0, _W)], tbs.at[s], semt).wait()
        pltpu.make_async_copy(th_hbm.at[:, pl.ds(0, _W)], tbh.at[s], semt).wait()

    def drain(n, sem):
        def d(t, carry):
            pltpu.make_async_copy(x_hbm.at[0], xrow_fb, sem).wait()
            return carry
        lax.fori_loop(0, n, d, 0)

    pltpu.sync_copy(tails_hbm, tails_v)
    pltpu.sync_copy(tailh_hbm, tailh_v)
    n_full = (_FULL_CHUNKS - 1 - wid) // _NW + 1
    fire_tables(0, 0, semt0)

    @pl.when(n_full > 1)
    def _():
        fire_tables(1, 1, semt1)

    for r in range(_SN // 16):
        skey[pl.ds(16 * r, 16)] = splat(jnp.int32(0x7FFFFFFF))

    def piece(p, ov):
        pltpu.sync_copy(idx_hbm.at[pl.ds(p * _PIECE, _PIECE)], idx_p)

        def scan(q, ov):
            v = plsc.load_gather(idx_p, [q * 16 + iota])
            m = ((v >> 7) & (_NW - 1)) == wid
            pos = ov + plsc.cumsum(jnp.where(m, 1, 0)) - 1
            ms = m & (pos < _WCAP)
            plsc.store_scatter(wb, [pos], p * _PIECE + q * 16 + iota, mask=ms)
            plsc.store_scatter(wi, [pos], v, mask=ms)
            plsc.store_scatter(skey, [pos], ((v >> 12) << 10) | pos, mask=ms)
            return ov + plsc.all_reduce_population_count(m)

        return lax.fori_loop(0, _PIECE // 16, scan, ov)

    m_true_v = lax.fori_loop(0, _NPIECE, piece, jnp.zeros((16,), jnp.int32))
    m_true = jnp.max(m_true_v)
    m_n = jnp.minimum(m_true, _WCAP)

    def vreg_sort_stage(kk):
        def vs(r, carry):
            v = plsc.load_gather(skey, [16 * r + iota])
            sv = lax.sort(v)
            asc = (r & (kk // 16)) == 0
            out = jnp.where(asc, sv, lax.rev(sv, (0,)))
            plsc.store_scatter(skey, [16 * r + iota], out)
            return carry
        lax.fori_loop(0, _SN // 16, vs, 0)

    vreg_sort_stage(16)
    for kk in (32, 64, 128, 256, 512, 1024):
        jv = kk // 32
        while jv >= 1:
            def xpass(p, carry, jv=jv, kk=kk):
                r = ((p & ~(jv - 1)) << 1) | (p & (jv - 1))
                a = plsc.load_gather(skey, [16 * r + iota])
                b = plsc.load_gather(skey, [16 * (r | jv) + iota])
                lo = jnp.minimum(a, b)
                hi = jnp.maximum(a, b)
                asc = (r & (kk // 16)) == 0
                plsc.store_scatter(skey, [16 * r + iota],
                                   jnp.where(asc, lo, hi))
                plsc.store_scatter(skey, [16 * (r | jv) + iota],
                                   jnp.where(asc, hi, lo))
                return carry
            lax.fori_loop(0, _SN // 32, xpass, 0)
            jv //= 2
        vreg_sort_stage(kk)

    def count_run(kl, ptr):
        def cond(st):
            return st[1]

        def bodyf(st):
            n, _ = st
            v = plsc.load_gather(skey, [splat(ptr + n) + iota])
            c = jnp.max(plsc.all_reduce_population_count((v >> 10) == kl))
            return (n + c, c == 16)

        n, _ = lax.while_loop(cond, bodyf, (jnp.int32(0), True))
        return n

    def fire_x(nm, ptr, bank, semx):
        def f(t, carry):
            mv = plsc.load_gather(skey, [splat(ptr + t)]) & 1023
            b = jnp.max(plsc.load_gather(wb, [mv]))
            pltpu.async_copy(x_hbm.at[b], xbank.at[bank, t], semx)
            return carry
        lax.fori_loop(0, nm, f, 0)

    def prefetch(kl, ptr, bank, semx):
        nt = count_run(kl, ptr)
        fire_x(jnp.minimum(nt, _MCAP), ptr, bank, semx)
        return nt

    def process(nm, ptr, bank, gather_sh, semo):
        def one(t, carry):
            mv = plsc.load_gather(skey, [splat(ptr + t)]) & 1023
            lc = plsc.load_gather(wi, [mv]) & (_W - 1)
            b = jnp.max(plsc.load_gather(wb, [mv]))
            for k in range(4):
                d = 16 * k + iota
                sv, hv = gather_sh(d, lc)
                xv = plsc.load_gather(xbank, [splat(bank), splat(t), d])
                plsc.store_scatter(obuf, [splat(bank), splat(t), d],
                                   xv * sv + hv)
            pltpu.async_copy(obuf.at[bank, t], out_hbm.at[b], semo)
            return carry
        lax.fori_loop(0, nm, one, 0)

    def run_chunk(nt, ptr, bank, gather_sh, semx, semo):
        nm0 = jnp.minimum(nt, _MCAP)
        drain(nm0, semx)
        process(nm0, ptr, bank, gather_sh, semo)

        def wave(w, prev):
            drain(prev, semo)
            nm_w = jnp.minimum(nt - w * _MCAP, _MCAP)
            fire_x(nm_w, ptr + w * _MCAP, bank, semx)
            drain(nm_w, semx)
            process(nm_w, ptr + w * _MCAP, bank, gather_sh, semo)
            return nm_w

        return lax.fori_loop(1, (nt + _MCAP - 1) // _MCAP, wave, nm0)

    def gsh_slot(s):
        def g(d, lc):
            return (plsc.load_gather(tbs, [splat(s), d, lc]),
                    plsc.load_gather(tbh, [splat(s), d, lc]))
        return g

    def gsh_tail(d, lc):
        return (plsc.load_gather(tails_v, [d, lc]),
                plsc.load_gather(tailh_v, [d, lc]))

    nt0 = prefetch(0, 0, 0, semx0)
    nt1 = jnp.int32(0)

    def step(g, carry):
        nt0, nt1, pt0, pt1, pp, p0, p1 = carry
        for s in (0, 1):
            k = 2 * g + s
            semt, semx, semo = ((semt0, semx0, semo0) if s == 0
                                else (semt1, semx1, semo1))
            nsemx = semx1 if s == 0 else semx0

            nt_next = lax.cond(
                k + 1 < n_full,
                lambda: prefetch(k + 1, pp, 1 - s, nsemx),
                lambda: jnp.int32(0))
            pp_next = pp + nt_next

            def active():
                wait_tables(s, semt)
                drain(p0 if s == 0 else p1, semo)
                fired = run_chunk(nt0 if s == 0 else nt1,
                                  pt0 if s == 0 else pt1,
                                  s, gsh_slot(s), semx, semo)

                @pl.when(k + 2 < n_full)
                def _():
                    fire_tables(k + 2, s, semt)

                return fired

            p_new = lax.cond(k < n_full, active, lambda: jnp.int32(0))
            if s == 0:
                p0, nt1, pt1 = p_new, nt_next, pp
            else:
                p1, nt0, pt0 = p_new, nt_next, pp
            pp = pp_next
        return (nt0, nt1, pt0, pt1, pp, p0, p1)

    _, _, _, _, pp, p0, p1 = lax.fori_loop(
        0, (n_full + 1) // 2, step,
        (nt0, nt1, jnp.int32(0), jnp.int32(0), nt0, jnp.int32(0),
         jnp.int32(0)))
    drain(p0, semo0)
    drain(p1, semo1)

    @pl.when(wid == (_FULL_CHUNKS & (_NW - 1)))
    def _():
        nt = prefetch(n_full, pp, 0, semx0)
        pt = run_chunk(nt, pp, 0, gsh_tail, semx0, semo0)
        drain(pt, semo0)

    @pl.when(m_true > _WCAP)
    def _():
        def fb_one(b, i):
            pltpu.sync_copy(x_hbm.at[b], xrow_fb)
            ci = i >> 7
            lc = splat(i & (_W - 1))

            def emit(gather_sh):
                for k in range(4):
                    d = 16 * k + iota
                    sv, hv = gather_sh(d, lc)
                    xv = plsc.load_gather(xrow_fb, [d])
                    plsc.store_scatter(obuf, [splat(0), splat(0), d],
                                       xv * sv + hv)

            @pl.when(ci < _FULL_CHUNKS)
            def _():
                pltpu.sync_copy(ts_hbm.at[:, pl.ds(ci * _W, _W)], tbs.at[0])
                pltpu.sync_copy(th_hbm.at[:, pl.ds(ci * _W, _W)], tbh.at[0])
                emit(gsh_slot(0))

            @pl.when(ci == _FULL_CHUNKS)
            def _():
                emit(gsh_tail)

            pltpu.sync_copy(obuf.at[0, 0], out_hbm.at[b])

        def fb_piece(p, o):
            pltpu.sync_copy(idx_hbm.at[pl.ds(p * _PIECE, _PIECE)], idx_p)

            def scan(q, o):
                v = plsc.load_gather(idx_p, [q * 16 + iota])
                m = ((v >> 7) & (_NW - 1)) == wid
                mi = jnp.where(m, 1, 0)
                pos = o + plsc.cumsum(mi) - 1
                ovf = m & (pos >= _WCAP)

                @pl.when(jnp.sum(jnp.where(ovf, 1, 0)) > 0)
                def _():
                    for l in range(16):
                        sel = jnp.where(iota == l, 1, 0)

                        @pl.when(jnp.sum(sel * jnp.where(ovf, 1, 0)) > 0)
                        def _():
                            fb_one(p * _PIECE + q * 16 + l, jnp.sum(sel * v))

                return o + jnp.sum(mi)

            return lax.fori_loop(0, _PIECE // 16, scan, o)

        lax.fori_loop(0, _NPIECE, fb_piece, 0)


@jax.jit
def _gauge(x, idx, tile_scale, tile_shift):
    ts = tile_scale.T
    th = tile_shift.T
    tails = ts[:, _TAIL_LO:]
    tailh = th[:, _TAIL_LO:]
    mesh = plsc.VectorSubcoreMesh(core_axis_name="c", subcore_axis_name="s")
    return pl.kernel(
        _body,
        mesh=mesh,
        compiler_params=pltpu.CompilerParams(
            use_tc_tiling_on_sc=True, needs_layout_passes=False),
        out_type=jax.ShapeDtypeStruct((BATCH, D_MODEL), jnp.float32),
        scratch_types=[
            pltpu.VMEM((_PIECE,), jnp.int32),
            pltpu.VMEM((_WCAP,), jnp.int32),
            pltpu.VMEM((_WCAP,), jnp.int32),
            pltpu.VMEM((_SN,), jnp.int32),
            pltpu.VMEM((2, D_MODEL, _W), jnp.float32),
            pltpu.VMEM((2, D_MODEL, _W), jnp.float32),
            pltpu.VMEM((D_MODEL, _NTAIL), jnp.float32),
            pltpu.VMEM((D_MODEL, _NTAIL), jnp.float32),
            pltpu.VMEM((2, _MCAP, D_MODEL), jnp.float32),
            pltpu.VMEM((2, _MCAP, D_MODEL), jnp.float32),
            pltpu.VMEM((D_MODEL,), jnp.float32),
            pltpu.SemaphoreType.DMA,
            pltpu.SemaphoreType.DMA,
            pltpu.SemaphoreType.DMA,
            pltpu.SemaphoreType.DMA,
            pltpu.SemaphoreType.DMA,
            pltpu.SemaphoreType.DMA,
        ],
    )(x, idx, ts, th, tails, tailh)


def kernel(x, tile_idx, tile_scale, tile_shift):
    return _gauge(x, tile_idx.astype(jnp.int32), tile_scale, tile_shift)


# speedup vs baseline: 1.2036x; 1.2036x over previous
import jax
import jax.numpy as jnp
from jax import lax
from jax.experimental import pallas as pl
from jax.experimental.pallas import tpu as pltpu
from jax.experimental.pallas import tpu_sc as plsc

NUM_TILES = 1000000
D_MODEL = 64
BATCH = 16384

_NW = 32
_W = 128
_FULL_CHUNKS = 7812
_TAIL_LO = _FULL_CHUNKS * _W
_NTAIL = NUM_TILES - _TAIL_LO
_WCAP = 960
_SN = 1040
_MCAP = 16
_PIECE = 4096
_NPIECE = BATCH // _PIECE


def _body(x_hbm, idx_hbm, ts_hbm, th_hbm, tails_hbm, tailh_hbm, out_hbm,
          idx_p, wb, wi, skey, tbs, tbh, xbank, obuf, xrow_fb,
          semt0, semt1, semt2, semx0, semx1, semx2, semo0, semo1, semo2):
    wid = lax.axis_index("s") * 2 + lax.axis_index("c")
    iota = lax.iota(jnp.int32, 16)

    def splat(v):
        return jnp.full((16,), v, jnp.int32)

    def fire_tables(k, s, semt):
        lo = (wid + _NW * k) * _W
        pltpu.async_copy(ts_hbm.at[:, pl.ds(lo, _W)], tbs.at[s], semt)
        pltpu.async_copy(th_hbm.at[:, pl.ds(lo, _W)], tbh.at[s], semt)

    def wait_tables(s, semt):
        pltpu.make_async_copy(ts_hbm.at[:, pl.ds(0, _W)], tbs.at[s], semt).wait()
        pltpu.make_async_copy(th_hbm.at[:, pl.ds(0, _W)], tbh.at[s], semt).wait()

    def drain(n, sem):
        def d(t, carry):
            pltpu.make_async_copy(x_hbm.at[0], xrow_fb, sem).wait()
            return carry
        lax.fori_loop(0, n, d, 0)

    n_full = (_FULL_CHUNKS - 1 - wid) // _NW + 1
    fire_tables(0, 0, semt0)
    fire_tables(1, 1, semt1)
    fire_tables(2, 2, semt2)

    for r in range(_SN // 16):
        skey[pl.ds(16 * r, 16)] = splat(jnp.int32(0x7FFFFFFF))

    def piece(p, ov):
        pltpu.sync_copy(idx_hbm.at[pl.ds(p * _PIECE, _PIECE)], idx_p)

        def scan(q, ov):
            v = plsc.load_gather(idx_p, [q * 16 + iota])
            m = ((v >> 7) & (_NW - 1)) == wid
            pos = ov + plsc.cumsum(jnp.where(m, 1, 0)) - 1
            ms = m & (pos < _WCAP)
            plsc.store_scatter(wb, [pos], p * _PIECE + q * 16 + iota, mask=ms)
            plsc.store_scatter(wi, [pos], v, mask=ms)
            plsc.store_scatter(skey, [pos], ((v >> 12) << 10) | pos, mask=ms)
            return ov + plsc.all_reduce_population_count(m)

        return lax.fori_loop(0, _PIECE // 16, scan, ov)

    m_true_v = lax.fori_loop(0, _NPIECE, piece, jnp.zeros((16,), jnp.int32))
    m_true = jnp.max(m_true_v)
    m_n = jnp.minimum(m_true, _WCAP)

    def vreg_sort_stage(kk):
        def vs(r, carry):
            v = plsc.load_gather(skey, [16 * r + iota])
            sv = lax.sort(v)
            asc = (r & (kk // 16)) == 0
            out = jnp.where(asc, sv, lax.rev(sv, (0,)))
            plsc.store_scatter(skey, [16 * r + iota], out)
            return carry
        lax.fori_loop(0, _SN // 16, vs, 0)

    vreg_sort_stage(16)
    for kk in (32, 64, 128, 256, 512, 1024):
        jv = kk // 32
        while jv >= 1:
            def xpass(p, carry, jv=jv, kk=kk):
                r = ((p & ~(jv - 1)) << 1) | (p & (jv - 1))
                a = plsc.load_gather(skey, [16 * r + iota])
                b = plsc.load_gather(skey, [16 * (r | jv) + iota])
                lo = jnp.minimum(a, b)
                hi = jnp.maximum(a, b)
                asc = (r & (kk // 16)) == 0
                plsc.store_scatter(skey, [16 * r + iota],
                                   jnp.where(asc, lo, hi))
                plsc.store_scatter(skey, [16 * (r | jv) + iota],
                                   jnp.where(asc, hi, lo))
                return carry
            lax.fori_loop(0, _SN // 32, xpass, 0)
            jv //= 2
        vreg_sort_stage(kk)

    def count_run(kl, ptr):
        def cond(st):
            return st[1]

        def bodyf(st):
            n, _ = st
            v = plsc.load_gather(skey, [splat(ptr + n) + iota])
            c = jnp.max(plsc.all_reduce_population_count((v >> 10) == kl))
            return (n + c, c == 16)

        n, _ = lax.while_loop(cond, bodyf, (jnp.int32(0), True))
        return n

    def fire_x(nm, ptr, bank, semx):
        def f(t, carry):
            mv = plsc.load_gather(skey, [splat(ptr + t)]) & 1023
            b = jnp.max(plsc.load_gather(wb, [mv]))
            pltpu.async_copy(x_hbm.at[b], xbank.at[bank, t], semx)
            return carry
        lax.fori_loop(0, nm, f, 0)

    def prefetch(kl, ptr, bank, semx):
        nt = count_run(kl, ptr)
        fire_x(jnp.minimum(nt, _MCAP), ptr, bank, semx)
        return nt

    def process(nm, ptr, bank, gather_sh, semo):
        def one(t, carry):
            mv = plsc.load_gather(skey, [splat(ptr + t)]) & 1023
            lc = plsc.load_gather(wi, [mv]) & (_W - 1)
            b = jnp.max(plsc.load_gather(wb, [mv]))
            for k in range(4):
                d = 16 * k + iota
                sv, hv = gather_sh(d, lc)
                xv = plsc.load_gather(xbank, [splat(bank), splat(t), d])
                plsc.store_scatter(obuf, [splat(bank), splat(t), d],
                                   xv * sv + hv)
            pltpu.async_copy(obuf.at[bank, t], out_hbm.at[b], semo)
            return carry
        lax.fori_loop(0, nm, one, 0)

    def run_chunk(nt, ptr, bank, gather_sh, semx, semo):
        nm0 = jnp.minimum(nt, _MCAP)
        drain(nm0, semx)
        process(nm0, ptr, bank, gather_sh, semo)

        def wave(w, prev):
            drain(prev, semo)
            nm_w = jnp.minimum(nt - w * _MCAP, _MCAP)
            fire_x(nm_w, ptr + w * _MCAP, bank, semx)
            drain(nm_w, semx)
            process(nm_w, ptr + w * _MCAP, bank, gather_sh, semo)
            return nm_w

        return lax.fori_loop(1, (nt + _MCAP - 1) // _MCAP, wave, nm0)

    def gsh_slot(s):
        def g(d, lc):
            return (plsc.load_gather(tbs, [splat(s), d, lc]),
                    plsc.load_gather(tbh, [splat(s), d, lc]))
        return g

    semts = (semt0, semt1, semt2)
    semxs = (semx0, semx1, semx2)
    semos = (semo0, semo1, semo2)
    nt_first = prefetch(0, 0, 0, semx0)
    z = jnp.int32(0)

    def step(g, carry):
        nts, pts, pp, ps = [list(carry[0:3]), list(carry[3:6]), carry[6],
                            list(carry[7:10])]
        for s in (0, 1, 2):
            k = 3 * g + s
            ns = (s + 1) % 3

            nt_next = lax.cond(
                k + 1 < n_full,
                lambda: prefetch(k + 1, pp, ns, semxs[ns]),
                lambda: jnp.int32(0))
            pp_next = pp + nt_next

            def active():
                wait_tables(s, semts[s])
                drain(ps[s], semos[s])
                fired = run_chunk(nts[s], pts[s], s, gsh_slot(s),
                                  semxs[s], semos[s])

                @pl.when(k + 3 < n_full)
                def _():
                    fire_tables(k + 3, s, semts[s])

                return fired

            ps[s] = lax.cond(k < n_full, active, lambda: jnp.int32(0))
            nts[ns] = nt_next
            pts[ns] = pp
            pp = pp_next
        return (*nts, *pts, pp, *ps)

    res = lax.fori_loop(0, (n_full + 2) // 3, step,
                        (nt_first, z, z, z, z, z, nt_first, z, z, z))
    pp, p0, p1, p2 = res[6], res[7], res[8], res[9]
    drain(p0, semo0)
    drain(p1, semo1)
    drain(p2, semo2)

    @pl.when(wid == (_FULL_CHUNKS & (_NW - 1)))
    def _():
        pltpu.sync_copy(tails_hbm, tbs.at[0])
        pltpu.sync_copy(tailh_hbm, tbh.at[0])
        nt = prefetch(n_full, pp, 0, semx0)
        pt = run_chunk(nt, pp, 0, gsh_slot(0), semx0, semo0)
        drain(pt, semo0)

    @pl.when(m_true > _WCAP)
    def _():
        def fb_one(b, i):
            pltpu.sync_copy(x_hbm.at[b], xrow_fb)
            ci = i >> 7
            lc = splat(i & (_W - 1))

            def emit(gather_sh):
                for k in range(4):
                    d = 16 * k + iota
                    sv, hv = gather_sh(d, lc)
                    xv = plsc.load_gather(xrow_fb, [d])
                    plsc.store_scatter(obuf, [splat(0), splat(0), d],
                                       xv * sv + hv)

            @pl.when(ci < _FULL_CHUNKS)
            def _():
                pltpu.sync_copy(ts_hbm.at[:, pl.ds(ci * _W, _W)], tbs.at[0])
                pltpu.sync_copy(th_hbm.at[:, pl.ds(ci * _W, _W)], tbh.at[0])
                emit(gsh_slot(0))

            @pl.when(ci == _FULL_CHUNKS)
            def _():
                pltpu.sync_copy(tails_hbm, tbs.at[0])
                pltpu.sync_copy(tailh_hbm, tbh.at[0])
                emit(gsh_slot(0))

            pltpu.sync_copy(obuf.at[0, 0], out_hbm.at[b])

        def fb_piece(p, o):
            pltpu.sync_copy(idx_hbm.at[pl.ds(p * _PIECE, _PIECE)], idx_p)

            def scan(q, o):
                v = plsc.load_gather(idx_p, [q * 16 + iota])
                m = ((v >> 7) & (_NW - 1)) == wid
                mi = jnp.where(m, 1, 0)
                pos = o + plsc.cumsum(mi) - 1
                ovf = m & (pos >= _WCAP)

                @pl.when(jnp.sum(jnp.where(ovf, 1, 0)) > 0)
                def _():
                    for l in range(16):
                        sel = jnp.where(iota == l, 1, 0)

                        @pl.when(jnp.sum(sel * jnp.where(ovf, 1, 0)) > 0)
                        def _():
                            fb_one(p * _PIECE + q * 16 + l, jnp.sum(sel * v))

                return o + jnp.sum(mi)

            return lax.fori_loop(0, _PIECE // 16, scan, o)

        lax.fori_loop(0, _NPIECE, fb_piece, 0)


@jax.jit
def _gauge(x, idx, tile_scale, tile_shift):
    ts = tile_scale.T
    th = tile_shift.T
    tails = jnp.pad(ts[:, _TAIL_LO:], ((0, 0), (0, _W - _NTAIL)))
    tailh = jnp.pad(th[:, _TAIL_LO:], ((0, 0), (0, _W - _NTAIL)))
    mesh = plsc.VectorSubcoreMesh(core_axis_name="c", subcore_axis_name="s")
    return pl.kernel(
        _body,
        mesh=mesh,
        compiler_params=pltpu.CompilerParams(
            use_tc_tiling_on_sc=True, needs_layout_passes=False),
        out_type=jax.ShapeDtypeStruct((BATCH, D_MODEL), jnp.float32),
        scratch_types=[
            pltpu.VMEM((_PIECE,), jnp.int32),
            pltpu.VMEM((_WCAP,), jnp.int32),
            pltpu.VMEM((_WCAP,), jnp.int32),
            pltpu.VMEM((_SN,), jnp.int32),
            pltpu.VMEM((3, D_MODEL, _W), jnp.float32),
            pltpu.VMEM((3, D_MODEL, _W), jnp.float32),
            pltpu.VMEM((3, _MCAP, D_MODEL), jnp.float32),
            pltpu.VMEM((3, _MCAP, D_MODEL), jnp.float32),
            pltpu.VMEM((D_MODEL,), jnp.float32),
            pltpu.SemaphoreType.DMA,
            pltpu.SemaphoreType.DMA,
            pltpu.SemaphoreType.DMA,
            pltpu.SemaphoreType.DMA,
            pltpu.SemaphoreType.DMA,
            pltpu.SemaphoreType.DMA,
            pltpu.SemaphoreType.DMA,
            pltpu.SemaphoreType.DMA,
            pltpu.SemaphoreType.DMA,
        ],
    )(x, idx, ts, th, tails, tailh)


def kernel(x, tile_idx, tile_scale, tile_shift):
    return _gauge(x, tile_idx.astype(jnp.int32), tile_scale, tile_shift)
